# Initial kernel scaffold; baseline (speedup 1.0000x reference)
import functools

import jax
import jax.numpy as jnp
from jax import lax
from jax.experimental import pallas as pl
from jax.experimental.pallas import tpu as pltpu
from jax.experimental.pallas import tpu_sc as plsc

N_NODES = 10000
D_FEAT = 128
N_EDGES = 320000
NUM_PASSES = 2
EPS = 1e-5

NC = 2
NS = 16
NW = NC * NS
LANES = 16

E_PER_W = N_EDGES // NW
K = 80
NCHUNK = E_PER_W // K
ROWS_PER_S = N_NODES // NS


def _sc_mesh():
    return plsc.VectorSubcoreMesh(core_axis_name="c", subcore_axis_name="s")


def _scatter_pass(x, src, dst, ew, zeros):
    @functools.partial(
        pl.kernel,
        out_type=jax.ShapeDtypeStruct((NC, N_NODES, D_FEAT), jnp.float32),
        mesh=_sc_mesh(),
        scratch_types=[
            pltpu.VMEM_SHARED((N_NODES, D_FEAT), jnp.float32),
            pltpu.VMEM((K,), jnp.int32),
            pltpu.VMEM((K,), jnp.int32),
            pltpu.VMEM((K,), jnp.float32),
            pltpu.VMEM((K, D_FEAT), jnp.float32),
        ],
    )
    def body(x_hbm, src_hbm, dst_hbm, ew_hbm, zeros_hbm, out_hbm,
             acc, src_v, dst_v, ew_v, rows_v):
        c = lax.axis_index("c")
        s = lax.axis_index("s")
        wid = s * NC + c

        row0 = s * ROWS_PER_S
        pltpu.sync_copy(zeros_hbm.at[pl.ds(row0, ROWS_PER_S)],
                        acc.at[pl.ds(row0, ROWS_PER_S)])
        plsc.subcore_barrier()

        @pl.loop(0, NCHUNK)
        def _(i):
            pltpu.sync_copy(src_hbm.at[wid, i], src_v)
            pltpu.sync_copy(dst_hbm.at[wid, i], dst_v)
            pltpu.sync_copy(ew_hbm.at[wid, i], ew_v)
            pltpu.sync_copy(x_hbm.at[src_v], rows_v)

            @pl.loop(0, K)
            def _(r):
                ridx = jnp.full((LANES,), r, dtype=jnp.int32)
                w = plsc.load_gather(ew_v, [ridx])

                @pl.loop(0, D_FEAT, step=LANES)
                def _(j):
                    rows_v[r, pl.ds(j, LANES)] = rows_v[r, pl.ds(j, LANES)] * w

            pltpu.sync_copy(rows_v, acc.at[dst_v], add=True)

        plsc.subcore_barrier()
        pltpu.sync_copy(acc.at[pl.ds(row0, ROWS_PER_S)],
                        out_hbm.at[c, pl.ds(row0, ROWS_PER_S)])

    return body(x, src, dst, ew, zeros)


def _log1p_body(w_ref, o_ref):
    o_ref[...] = jnp.log1p(w_ref[...])


def _log1p_tc(w2d):
    return pl.pallas_call(
        _log1p_body,
        out_shape=jax.ShapeDtypeStruct(w2d.shape, jnp.float32),
    )(w2d)


def _combine_ln_body(p_ref, w_ref, b_ref, o_ref):
    x = p_ref[0] + p_ref[1]
    mean = jnp.mean(x, axis=0, keepdims=True)
    xm = x - mean
    var = jnp.mean(xm * xm, axis=0, keepdims=True)
    inv = lax.rsqrt(var + EPS)
    o_ref[...] = xm * inv * w_ref[...] + b_ref[...]


def _combine_ln_tc(parts, ln_w, ln_b):
    return pl.pallas_call(
        _combine_ln_body,
        out_shape=jax.ShapeDtypeStruct((N_NODES, D_FEAT), jnp.float32),
    )(parts, ln_w, ln_b)


def kernel(x, edge_index, edge_weight, ln_weight, ln_bias):
    src = edge_index[0].astype(jnp.int32).reshape(NW, NCHUNK, K)
    dst = edge_index[1].astype(jnp.int32).reshape(NW, NCHUNK, K)
    ew = _log1p_tc(edge_weight.reshape(2500, 128)).reshape(NW, NCHUNK, K)
    zeros = jnp.zeros((N_NODES, D_FEAT), jnp.float32)
    ln_w = ln_weight.reshape(N_NODES, 1)
    ln_b = ln_bias.reshape(N_NODES, 1)
    for _ in range(NUM_PASSES):
        parts = _scatter_pass(x, src, dst, ew, zeros)
        x = _combine_ln_tc(parts, ln_w, ln_b)
    return x


# speedup vs baseline: 3.6402x; 3.6402x over previous
import dataclasses
import functools

import jax
import jax.numpy as jnp
from jax import lax
from jax.experimental import pallas as pl
from jax.experimental.pallas import tpu as pltpu
from jax.experimental.pallas import tpu_sc as plsc

N_NODES = 10000
D_FEAT = 128
N_EDGES = 320000
NUM_PASSES = 2
EPS = 1e-5

NC = 2
NS = 16
NW = NC * NS
LANES = 16

E_PER_W = N_EDGES // NW
K = 80
NCHUNK = E_PER_W // K
N_PAD = 10240
ROWS_PER_S = N_PAD // NS


def _sc_mesh():
    return plsc.VectorSubcoreMesh(core_axis_name="c", subcore_axis_name="s")


def _sc_compiler_params():
    cp = pltpu.CompilerParams()
    if "needs_layout_passes" in pltpu.CompilerParams.__dataclass_fields__:
        cp = dataclasses.replace(cp, needs_layout_passes=False)
    return cp


def _scatter_pass(x, src, dst, ew, zeros):
    @functools.partial(
        pl.kernel,
        out_type=jax.ShapeDtypeStruct((NC, N_PAD, D_FEAT), jnp.float32),
        mesh=_sc_mesh(),
        compiler_params=_sc_compiler_params(),
        scratch_types=[
            pltpu.VMEM_SHARED((N_PAD, D_FEAT), jnp.float32),
            pltpu.VMEM((K,), jnp.int32),
            pltpu.VMEM((K,), jnp.int32),
            pltpu.VMEM((K,), jnp.float32),
            pltpu.VMEM((K, D_FEAT), jnp.float32),
        ],
    )
    def body(x_hbm, src_hbm, dst_hbm, ew_hbm, zeros_hbm, out_hbm,
             acc, src_v, dst_v, ew_v, rows_v):
        c = lax.axis_index("c")
        s = lax.axis_index("s")
        wid = s * NC + c

        row0 = s * ROWS_PER_S
        pltpu.sync_copy(zeros_hbm.at[pl.ds(row0, ROWS_PER_S)],
                        acc.at[pl.ds(row0, ROWS_PER_S)])
        plsc.subcore_barrier()

        @pl.loop(0, NCHUNK)
        def _(i):
            pltpu.sync_copy(src_hbm.at[wid, i], src_v)
            pltpu.sync_copy(dst_hbm.at[wid, i], dst_v)
            pltpu.sync_copy(ew_hbm.at[wid, i], ew_v)
            pltpu.sync_copy(x_hbm.at[src_v], rows_v)

            @pl.loop(0, K)
            def _(r):
                ridx = jnp.full((LANES,), r, dtype=jnp.int32)
                w = plsc.load_gather(ew_v, [ridx])

                @pl.loop(0, D_FEAT, step=LANES)
                def _(j):
                    rows_v[r, pl.ds(j, LANES)] = rows_v[r, pl.ds(j, LANES)] * w

            pltpu.sync_copy(rows_v, acc.at[dst_v], add=True)

        plsc.subcore_barrier()
        pltpu.sync_copy(acc.at[pl.ds(row0, ROWS_PER_S)],
                        out_hbm.at[c, pl.ds(row0, ROWS_PER_S)])

    return body(x, src, dst, ew, zeros)


def _log1p_body(w_ref, o_ref):
    o_ref[...] = jnp.log1p(w_ref[...])


def _log1p_tc(w2d):
    return pl.pallas_call(
        _log1p_body,
        out_shape=jax.ShapeDtypeStruct(w2d.shape, jnp.float32),
    )(w2d)


def _combine_ln_body(p_ref, w_ref, b_ref, o_ref):
    x = p_ref[0, :N_NODES] + p_ref[1, :N_NODES]
    mean = jnp.mean(x, axis=0, keepdims=True)
    xm = x - mean
    var = jnp.mean(xm * xm, axis=0, keepdims=True)
    inv = lax.rsqrt(var + EPS)
    o_ref[...] = xm * inv * w_ref[...] + b_ref[...]


def _combine_ln_tc(parts, ln_w, ln_b):
    return pl.pallas_call(
        _combine_ln_body,
        out_shape=jax.ShapeDtypeStruct((N_NODES, D_FEAT), jnp.float32),
    )(parts, ln_w, ln_b)


def kernel(x, edge_index, edge_weight, ln_weight, ln_bias):
    src = edge_index[0].astype(jnp.int32).reshape(NW, NCHUNK, K)
    dst = edge_index[1].astype(jnp.int32).reshape(NW, NCHUNK, K)
    ew = _log1p_tc(edge_weight.reshape(2500, 128)).reshape(NW, NCHUNK, K)
    zeros = jnp.zeros((N_PAD, D_FEAT), jnp.float32)
    ln_w = ln_weight.reshape(N_NODES, 1)
    ln_b = ln_bias.reshape(N_NODES, 1)
    for _ in range(NUM_PASSES):
        parts = _scatter_pass(x, src, dst, ew, zeros)
        x = _combine_ln_tc(parts, ln_w, ln_b)
    return x


# speedup vs baseline: 4.2420x; 1.1653x over previous
import dataclasses
import functools

import jax
import jax.numpy as jnp
from jax import lax
from jax.experimental import pallas as pl
from jax.experimental.pallas import tpu as pltpu
from jax.experimental.pallas import tpu_sc as plsc

N_NODES = 10000
D_FEAT = 128
N_EDGES = 320000
NUM_PASSES = 2
EPS = 1e-5

NC = 2
NS = 16
NW = NC * NS
LANES = 16

E_PER_W = N_EDGES // NW
K = 128
NCHUNK = -(-E_PER_W // K)
E_PAD_W = NCHUNK * K
N_PAD = 10240
ROWS_PER_S = N_PAD // NS


def _sc_mesh():
    return plsc.VectorSubcoreMesh(core_axis_name="c", subcore_axis_name="s")


def _sc_compiler_params():
    cp = pltpu.CompilerParams()
    if "needs_layout_passes" in pltpu.CompilerParams.__dataclass_fields__:
        cp = dataclasses.replace(cp, needs_layout_passes=False)
    return cp


def _scatter_pass(x, src, dst, ew, zeros):
    @functools.partial(
        pl.kernel,
        out_type=jax.ShapeDtypeStruct((NC, N_PAD, D_FEAT), jnp.float32),
        mesh=_sc_mesh(),
        compiler_params=_sc_compiler_params(),
        scratch_types=[
            pltpu.VMEM_SHARED((N_PAD, D_FEAT), jnp.float32),
            pltpu.VMEM((NCHUNK, K), jnp.int32),
            pltpu.VMEM((NCHUNK, K), jnp.int32),
            pltpu.VMEM((NCHUNK, K), jnp.float32),
            pltpu.VMEM((K, D_FEAT), jnp.float32),
        ],
    )
    def body(x_hbm, src_hbm, dst_hbm, ew_hbm, zeros_hbm, out_hbm,
             acc, src_v, dst_v, ew_v, rows_v):
        c = lax.axis_index("c")
        s = lax.axis_index("s")
        wid = s * NC + c

        row0 = s * ROWS_PER_S
        pltpu.sync_copy(zeros_hbm.at[pl.ds(row0, ROWS_PER_S)],
                        acc.at[pl.ds(row0, ROWS_PER_S)])
        pltpu.sync_copy(src_hbm.at[wid], src_v)
        pltpu.sync_copy(dst_hbm.at[wid], dst_v)
        pltpu.sync_copy(ew_hbm.at[wid], ew_v)
        plsc.subcore_barrier()

        @pl.loop(0, NCHUNK)
        def _(i):
            pltpu.sync_copy(x_hbm.at[src_v.at[i]], rows_v)

            @pl.loop(0, K)
            def _(r):
                iidx = jnp.full((LANES,), i, dtype=jnp.int32)
                ridx = jnp.full((LANES,), r, dtype=jnp.int32)
                w = plsc.load_gather(ew_v, [iidx, ridx])
                for j in range(0, D_FEAT, LANES):
                    rows_v[r, pl.ds(j, LANES)] = rows_v[r, pl.ds(j, LANES)] * w

            pltpu.sync_copy(rows_v, acc.at[dst_v.at[i]], add=True)

        plsc.subcore_barrier()
        pltpu.sync_copy(acc.at[pl.ds(row0, ROWS_PER_S)],
                        out_hbm.at[c, pl.ds(row0, ROWS_PER_S)])

    return body(x, src, dst, ew, zeros)


def _log1p_body(w_ref, o_ref):
    o_ref[...] = jnp.log1p(w_ref[...])


def _log1p_tc(w2d):
    return pl.pallas_call(
        _log1p_body,
        out_shape=jax.ShapeDtypeStruct(w2d.shape, jnp.float32),
    )(w2d)


def _combine_ln_body(p_ref, w_ref, b_ref, o_ref):
    x = p_ref[0, :N_NODES] + p_ref[1, :N_NODES]
    mean = jnp.mean(x, axis=0, keepdims=True)
    xm = x - mean
    var = jnp.mean(xm * xm, axis=0, keepdims=True)
    inv = lax.rsqrt(var + EPS)
    o_ref[...] = xm * inv * w_ref[...] + b_ref[...]


def _combine_ln_tc(parts, ln_w, ln_b):
    return pl.pallas_call(
        _combine_ln_body,
        out_shape=jax.ShapeDtypeStruct((N_NODES, D_FEAT), jnp.float32),
    )(parts, ln_w, ln_b)


def _pad_edges(a, fill):
    a = a.reshape(NW, E_PER_W)
    a = jnp.pad(a, ((0, 0), (0, E_PAD_W - E_PER_W)), constant_values=fill)
    return a.reshape(NW, NCHUNK, K)


def kernel(x, edge_index, edge_weight, ln_weight, ln_bias):
    src = _pad_edges(edge_index[0].astype(jnp.int32), 0)
    dst = _pad_edges(edge_index[1].astype(jnp.int32), N_PAD - 1)
    ew = _pad_edges(_log1p_tc(edge_weight.reshape(2500, 128)).reshape(-1), 0.0)
    zeros = jnp.zeros((N_PAD, D_FEAT), jnp.float32)
    ln_w = ln_weight.reshape(N_NODES, 1)
    ln_b = ln_bias.reshape(N_NODES, 1)
    for _ in range(NUM_PASSES):
        parts = _scatter_pass(x, src, dst, ew, zeros)
        x = _combine_ln_tc(parts, ln_w, ln_b)
    return x
